# baseline (speedup 1.0000x reference)
import functools

import jax
import jax.numpy as jnp
from jax import lax
from jax.experimental import pallas as pl
from jax.experimental.pallas import tpu as pltpu
from jax.experimental.pallas import tpu_sc as plsc

NUM_EMBEDDINGS = 8192
EMBEDDING_DIM = 256
BETA = 0.25

N_TOKENS = 8192

TM = 1024
TK = 1024
N_I = N_TOKENS // TM
N_J = NUM_EMBEDDINGS // TK

CHUNK_SPLIT_TILE = (NUM_EMBEDDINGS // 2) // TK


def _bf16_round(x):
    return x.astype(jnp.bfloat16).astype(jnp.float32)


def _argmin_body(zsq_ref, zb2_ref, emb_ref, esq_ref, idx_ref, loss_ref,
                 v0, v1, a0, a1, loss_acc):
    j = pl.program_id(1)

    @pl.when(j == 0)
    def _init():
        v0[...] = jnp.full((TM, 1), jnp.inf, jnp.float32)
        v1[...] = jnp.full((TM, 1), jnp.inf, jnp.float32)
        a0[...] = jnp.zeros((TM, 1), jnp.int32)
        a1[...] = jnp.zeros((TM, 1), jnp.int32)

    dot = lax.dot_general(zb2_ref[...], emb_ref[...],
                          (((1,), (1,)), ((), ())),
                          preferred_element_type=jnp.float32)
    dist = (zsq_ref[...] - dot) + esq_ref[...]

    col = lax.broadcasted_iota(jnp.int32, (TM, TK), 1) + j * TK
    lmin = jnp.min(dist, axis=1, keepdims=True)
    larg = jnp.min(jnp.where(dist == lmin, col, NUM_EMBEDDINGS),
                   axis=1, keepdims=True)

    in0 = j < CHUNK_SPLIT_TILE
    upd0 = (lmin < v0[...]) & in0
    v0[...] = jnp.where(upd0, lmin, v0[...])
    a0[...] = jnp.where(upd0, larg, a0[...])
    upd1 = (lmin < v1[...]) & jnp.logical_not(in0)
    v1[...] = jnp.where(upd1, lmin, v1[...])
    a1[...] = jnp.where(upd1, larg, a1[...])

    @pl.when(j == N_J - 1)
    def _finish():
        r = _bf16_round(v0[...])
        u1 = v1[...] < r
        pick = jnp.where(u1, a1[...], a0[...])
        pickv = jnp.where(u1, v1[...], v0[...])

        idx_ref[...] = pick
        i = pl.program_id(0)
        part = jnp.sum(pickv)

        @pl.when(i == 0)
        def _():
            loss_acc[0] = part

        @pl.when(i > 0)
        def _():
            loss_acc[0] = loss_acc[0] + part

        @pl.when(i == N_I - 1)
        def _():
            loss_ref[...] = jnp.full((1, 1), loss_acc[0], jnp.float32)


def _dist_argmin(zsq, zb2, emb, esq):
    return pl.pallas_call(
        _argmin_body,
        grid=(N_I, N_J),
        in_specs=[
            pl.BlockSpec((TM, 1), lambda i, j: (i, 0)),
            pl.BlockSpec((TM, EMBEDDING_DIM), lambda i, j: (i, 0)),
            pl.BlockSpec((TK, EMBEDDING_DIM), lambda i, j: (j, 0)),
            pl.BlockSpec((1, TK), lambda i, j: (0, j)),
        ],
        out_specs=[
            pl.BlockSpec((TM, 1), lambda i, j: (i, 0)),
            pl.BlockSpec((1, 1), lambda i, j: (0, 0)),
        ],
        out_shape=[
            jax.ShapeDtypeStruct((N_TOKENS, 1), jnp.int32),
            jax.ShapeDtypeStruct((1, 1), jnp.float32),
        ],
        scratch_shapes=[
            pltpu.VMEM((TM, 1), jnp.float32),
            pltpu.VMEM((TM, 1), jnp.float32),
            pltpu.VMEM((TM, 1), jnp.int32),
            pltpu.VMEM((TM, 1), jnp.int32),
            pltpu.SMEM((1,), jnp.float32),
        ],
    )(zsq, zb2, emb, esq)


@functools.cache
def _make_sc_gather():
    info = plsc.get_sparse_core_info()
    nw = info.num_cores * info.num_subcores
    b_per_w = N_TOKENS // nw
    n_sub = b_per_w // 128
    mesh = plsc.VectorSubcoreMesh(core_axis_name="c", subcore_axis_name="s")

    @functools.partial(
        pl.kernel, mesh=mesh,
        out_type=jax.ShapeDtypeStruct((N_TOKENS, EMBEDDING_DIM), jnp.float32),
        scratch_types=[
            pltpu.VMEM((n_sub, 128), jnp.int32),
            pltpu.VMEM((b_per_w, EMBEDDING_DIM), jnp.float32),
            pltpu.SemaphoreType.DMA,
        ],
    )
    def gather(table_hbm, idx_hbm, out_hbm, idx_v, rows_v, sem):
        wid = lax.axis_index("s") * info.num_cores + lax.axis_index("c")
        base = wid * b_per_w
        for b in range(n_sub):
            pltpu.sync_copy(idx_hbm.at[pl.ds(base + b * 128, 128)], idx_v.at[b])
            pltpu.async_copy(table_hbm.at[idx_v.at[b]],
                             rows_v.at[pl.ds(b * 128, 128)], sem).wait()
        pltpu.sync_copy(rows_v, out_hbm.at[pl.ds(base, b_per_w)])

    return gather


def kernel(z, emb):
    B, C, H, W = z.shape
    z_flat = jnp.transpose(z, (0, 2, 3, 1)).reshape(-1, EMBEDDING_DIM)
    zb2 = (2.0 * z_flat).astype(jnp.bfloat16)
    zsq = jnp.sum(z_flat ** 2, axis=1, keepdims=True)
    esq = jnp.sum(emb ** 2, axis=1)[None, :]

    idx2d, loss_sum = _dist_argmin(zsq, zb2, emb, esq)
    idx = idx2d.reshape(-1)

    q_flat = _make_sc_gather()(emb, idx)

    quantized = jnp.transpose(q_flat.reshape(B, H, W, C), (0, 3, 1, 2))
    codebook_loss = loss_sum[0, 0] / jnp.float32(N_TOKENS * EMBEDDING_DIM)
    commitment_loss = BETA * codebook_loss
    quantized_straight_through = z + lax.stop_gradient(quantized - z)
    return (quantized_straight_through, commitment_loss, codebook_loss)


# speedup vs baseline: 1.3655x; 1.3655x over previous
import functools

import jax
import jax.numpy as jnp
from jax import lax
from jax.experimental import pallas as pl
from jax.experimental.pallas import tpu as pltpu
from jax.experimental.pallas import tpu_sc as plsc

NUM_EMBEDDINGS = 8192
EMBEDDING_DIM = 256
BETA = 0.25

N_TOKENS = 8192

TM = 1024
TK = 4096
N_I = N_TOKENS // TM
N_J = NUM_EMBEDDINGS // TK

CHUNK_SPLIT_TILE = (NUM_EMBEDDINGS // 2) // TK


def _bf16_round(x):
    return x.astype(jnp.bfloat16).astype(jnp.float32)


def _argmin_body(zsq_ref, zb2_ref, emb_ref, esq_ref, idx_ref, loss_ref,
                 v0, v1, a0, a1, loss_acc):
    j = pl.program_id(1)

    @pl.when(j == 0)
    def _init():
        v0[...] = jnp.full((TM, 1), jnp.inf, jnp.float32)
        v1[...] = jnp.full((TM, 1), jnp.inf, jnp.float32)
        a0[...] = jnp.zeros((TM, 1), jnp.int32)
        a1[...] = jnp.zeros((TM, 1), jnp.int32)

    dot = lax.dot_general(zb2_ref[...], emb_ref[...],
                          (((1,), (1,)), ((), ())),
                          preferred_element_type=jnp.float32)
    zsq = zsq_ref[...]

    NB = TK // 128
    mval = jnp.full((TM, 128), jnp.inf, jnp.float32)
    mblk = jnp.zeros((TM, 128), jnp.float32)
    for b in range(NB):
        dblk = ((zsq - dot[:, b * 128:(b + 1) * 128])
                + esq_ref[:, b * 128:(b + 1) * 128])
        cmp = dblk < mval
        mval = jnp.where(cmp, dblk, mval)
        mblk = jnp.where(cmp, jnp.float32(b), mblk)

    lane = lax.broadcasted_iota(jnp.int32, (TM, 128), 1).astype(jnp.float32)
    mcol = mblk * 128.0 + lane
    lmin = jnp.min(mval, axis=1, keepdims=True)
    larg_f = jnp.min(jnp.where(mval == lmin, mcol, jnp.float32(TK)),
                     axis=1, keepdims=True)
    larg = larg_f.astype(jnp.int32) + j * TK

    in0 = j < CHUNK_SPLIT_TILE
    upd0 = (lmin < v0[...]) & in0
    v0[...] = jnp.where(upd0, lmin, v0[...])
    a0[...] = jnp.where(upd0, larg, a0[...])
    upd1 = (lmin < v1[...]) & jnp.logical_not(in0)
    v1[...] = jnp.where(upd1, lmin, v1[...])
    a1[...] = jnp.where(upd1, larg, a1[...])

    @pl.when(j == N_J - 1)
    def _finish():
        r = _bf16_round(v0[...])
        u1 = v1[...] < r
        pick = jnp.where(u1, a1[...], a0[...])
        pickv = jnp.where(u1, v1[...], v0[...])

        idx_ref[...] = pick
        i = pl.program_id(0)
        part = jnp.sum(pickv)

        @pl.when(i == 0)
        def _():
            loss_acc[0] = part

        @pl.when(i > 0)
        def _():
            loss_acc[0] = loss_acc[0] + part

        @pl.when(i == N_I - 1)
        def _():
            loss_ref[...] = jnp.full((1, 1), loss_acc[0], jnp.float32)


def _dist_argmin(zsq, zb2, emb, esq):
    return pl.pallas_call(
        _argmin_body,
        grid=(N_I, N_J),
        in_specs=[
            pl.BlockSpec((TM, 1), lambda i, j: (i, 0)),
            pl.BlockSpec((TM, EMBEDDING_DIM), lambda i, j: (i, 0)),
            pl.BlockSpec((TK, EMBEDDING_DIM), lambda i, j: (j, 0)),
            pl.BlockSpec((1, TK), lambda i, j: (0, j)),
        ],
        out_specs=[
            pl.BlockSpec((TM, 1), lambda i, j: (i, 0)),
            pl.BlockSpec((1, 1), lambda i, j: (0, 0)),
        ],
        out_shape=[
            jax.ShapeDtypeStruct((N_TOKENS, 1), jnp.int32),
            jax.ShapeDtypeStruct((1, 1), jnp.float32),
        ],
        scratch_shapes=[
            pltpu.VMEM((TM, 1), jnp.float32),
            pltpu.VMEM((TM, 1), jnp.float32),
            pltpu.VMEM((TM, 1), jnp.int32),
            pltpu.VMEM((TM, 1), jnp.int32),
            pltpu.SMEM((1,), jnp.float32),
        ],
    )(zsq, zb2, emb, esq)


@functools.cache
def _make_sc_gather():
    info = plsc.get_sparse_core_info()
    nw = info.num_cores * info.num_subcores
    b_per_w = N_TOKENS // nw
    n_sub = b_per_w // 128
    mesh = plsc.VectorSubcoreMesh(core_axis_name="c", subcore_axis_name="s")

    @functools.partial(
        pl.kernel, mesh=mesh,
        out_type=jax.ShapeDtypeStruct((N_TOKENS, EMBEDDING_DIM), jnp.float32),
        scratch_types=[
            pltpu.VMEM((n_sub, 128), jnp.int32),
            pltpu.VMEM((b_per_w, EMBEDDING_DIM), jnp.float32),
            pltpu.SemaphoreType.DMA,
        ],
    )
    def gather(table_hbm, idx_hbm, out_hbm, idx_v, rows_v, sem):
        wid = lax.axis_index("s") * info.num_cores + lax.axis_index("c")
        base = wid * b_per_w
        for b in range(n_sub):
            pltpu.sync_copy(idx_hbm.at[pl.ds(base + b * 128, 128)], idx_v.at[b])
            pltpu.async_copy(table_hbm.at[idx_v.at[b]],
                             rows_v.at[pl.ds(b * 128, 128)], sem).wait()
        pltpu.sync_copy(rows_v, out_hbm.at[pl.ds(base, b_per_w)])

    return gather


def kernel(z, emb):
    B, C, H, W = z.shape
    z_flat = jnp.transpose(z, (0, 2, 3, 1)).reshape(-1, EMBEDDING_DIM)
    zb2 = (2.0 * z_flat).astype(jnp.bfloat16)
    zsq = jnp.sum(z_flat ** 2, axis=1, keepdims=True)
    esq = jnp.sum(emb ** 2, axis=1)[None, :]

    idx2d, loss_sum = _dist_argmin(zsq, zb2, emb, esq)
    idx = idx2d.reshape(-1)

    q_flat = _make_sc_gather()(emb, idx)

    quantized = jnp.transpose(q_flat.reshape(B, H, W, C), (0, 3, 1, 2))
    codebook_loss = loss_sum[0, 0] / jnp.float32(N_TOKENS * EMBEDDING_DIM)
    commitment_loss = BETA * codebook_loss
    quantized_straight_through = z + lax.stop_gradient(quantized - z)
    return (quantized_straight_through, commitment_loss, codebook_loss)


# speedup vs baseline: 1.4208x; 1.0406x over previous
import functools

import jax
import jax.numpy as jnp
from jax import lax
from jax.experimental import pallas as pl
from jax.experimental.pallas import tpu as pltpu
from jax.experimental.pallas import tpu_sc as plsc

NUM_EMBEDDINGS = 8192
EMBEDDING_DIM = 256
BETA = 0.25

N_TOKENS = 8192

TM = 1024
N_I = N_TOKENS // TM
HALF = NUM_EMBEDDINGS // 2


def _bf16_round(x):
    return x.astype(jnp.bfloat16).astype(jnp.float32)


def _half_argmin(zsq, dot, esq_ref, base):
    mval = jnp.full((TM, 128), jnp.inf, jnp.float32)
    mblk = jnp.zeros((TM, 128), jnp.float32)
    for b in range(HALF // 128):
        lo = base + b * 128
        dblk = (zsq - dot[:, lo:lo + 128]) + esq_ref[:, lo:lo + 128]
        cmp = dblk < mval
        mval = jnp.where(cmp, dblk, mval)
        mblk = jnp.where(cmp, jnp.float32(b), mblk)
    lane = lax.broadcasted_iota(jnp.int32, (TM, 128), 1).astype(jnp.float32)
    mcol = mblk * 128.0 + lane
    lmin = jnp.min(mval, axis=1, keepdims=True)
    larg_f = jnp.min(jnp.where(mval == lmin, mcol, jnp.float32(HALF)),
                     axis=1, keepdims=True)
    return lmin, larg_f.astype(jnp.int32) + base


def _argmin_body(zsq_ref, zb2_ref, emb_ref, esq_ref, idx_ref, loss_ref,
                 loss_acc):
    dot = lax.dot_general(zb2_ref[...], emb_ref[...],
                          (((1,), (1,)), ((), ())),
                          preferred_element_type=jnp.float32)
    zsq = zsq_ref[...]

    v0, a0 = _half_argmin(zsq, dot, esq_ref, 0)
    v1, a1 = _half_argmin(zsq, dot, esq_ref, HALF)

    r = _bf16_round(v0)
    u1 = v1 < r
    pick = jnp.where(u1, a1, a0)
    pickv = jnp.where(u1, v1, v0)

    idx_ref[...] = pick
    i = pl.program_id(0)
    part = jnp.sum(pickv)

    @pl.when(i == 0)
    def _():
        loss_acc[0] = part

    @pl.when(i > 0)
    def _():
        loss_acc[0] = loss_acc[0] + part

    @pl.when(i == N_I - 1)
    def _():
        loss_ref[...] = jnp.full((1, 1), loss_acc[0], jnp.float32)


def _dist_argmin(zsq, zb2, emb, esq):
    return pl.pallas_call(
        _argmin_body,
        grid=(N_I,),
        in_specs=[
            pl.BlockSpec((TM, 1), lambda i: (i, 0)),
            pl.BlockSpec((TM, EMBEDDING_DIM), lambda i: (i, 0)),
            pl.BlockSpec((NUM_EMBEDDINGS, EMBEDDING_DIM), lambda i: (0, 0)),
            pl.BlockSpec((1, NUM_EMBEDDINGS), lambda i: (0, 0)),
        ],
        out_specs=[
            pl.BlockSpec((TM, 1), lambda i: (i, 0)),
            pl.BlockSpec((1, 1), lambda i: (0, 0)),
        ],
        out_shape=[
            jax.ShapeDtypeStruct((N_TOKENS, 1), jnp.int32),
            jax.ShapeDtypeStruct((1, 1), jnp.float32),
        ],
        scratch_shapes=[
            pltpu.SMEM((1,), jnp.float32),
        ],
    )(zsq, zb2, emb, esq)


@functools.cache
def _make_sc_gather():
    info = plsc.get_sparse_core_info()
    nw = info.num_cores * info.num_subcores
    b_per_w = N_TOKENS // nw
    n_sub = b_per_w // 128
    mesh = plsc.VectorSubcoreMesh(core_axis_name="c", subcore_axis_name="s")

    @functools.partial(
        pl.kernel, mesh=mesh,
        out_type=jax.ShapeDtypeStruct((N_TOKENS, EMBEDDING_DIM), jnp.float32),
        scratch_types=[
            pltpu.VMEM((n_sub, 128), jnp.int32),
            pltpu.VMEM((b_per_w, EMBEDDING_DIM), jnp.float32),
            pltpu.SemaphoreType.DMA,
        ],
    )
    def gather(table_hbm, idx_hbm, out_hbm, idx_v, rows_v, sem):
        wid = lax.axis_index("s") * info.num_cores + lax.axis_index("c")
        base = wid * b_per_w
        for b in range(n_sub):
            pltpu.sync_copy(idx_hbm.at[pl.ds(base + b * 128, 128)], idx_v.at[b])
            pltpu.async_copy(table_hbm.at[idx_v.at[b]],
                             rows_v.at[pl.ds(b * 128, 128)], sem).wait()
        pltpu.sync_copy(rows_v, out_hbm.at[pl.ds(base, b_per_w)])

    return gather


def kernel(z, emb):
    B, C, H, W = z.shape
    z_flat = jnp.transpose(z, (0, 2, 3, 1)).reshape(-1, EMBEDDING_DIM)
    zb2 = (2.0 * z_flat).astype(jnp.bfloat16)
    zsq = jnp.sum(z_flat ** 2, axis=1, keepdims=True)
    esq = jnp.sum(emb ** 2, axis=1)[None, :]

    idx2d, loss_sum = _dist_argmin(zsq, zb2, emb, esq)
    idx = idx2d.reshape(-1)

    q_flat = _make_sc_gather()(emb, idx)

    quantized = jnp.transpose(q_flat.reshape(B, H, W, C), (0, 3, 1, 2))
    codebook_loss = loss_sum[0, 0] / jnp.float32(N_TOKENS * EMBEDDING_DIM)
    commitment_loss = BETA * codebook_loss
    quantized_straight_through = z + lax.stop_gradient(quantized - z)
    return (quantized_straight_through, commitment_loss, codebook_loss)
